# Initial kernel scaffold; baseline (speedup 1.0000x reference)
import jax
import jax.numpy as jnp
from jax.experimental import pallas as pl


def kernel(x, edge_index, W, b):
    raise NotImplementedError("write your pallas kernel here")


# speedup vs baseline: 7.6984x; 7.6984x over previous
import jax
import jax.numpy as jnp
from jax import lax
from jax.experimental import pallas as pl
from jax.experimental.pallas import tpu as pltpu
from jax.experimental.pallas import tpu_sc as plsc

N_NODES = 10000
N_EDGES = 320000
D = 128
OUT = 128
BOUND = 10.0

_NC = 2
_NS = 16
_NW = _NC * _NS
_E_TILE = N_EDGES // _NW
_CHUNK = 80
_NCHUNK = _E_TILE // _CHUNK
_ZROWS = 125
_ROWS_TILE = N_NODES // _NS


def _sc_agg_body(x_hbm, src_hbm, dst_hbm, out_hbm,
                 src_v, dst_v, rows_v, zbuf, agg_sh, sem):
    c = lax.axis_index("c")
    s = lax.axis_index("s")
    tile = c * _NS + s

    zvec = jnp.zeros((16,), jnp.float32)

    def _zero_elem(k, _):
        i = k // (D // 16)
        j = k % (D // 16)
        zbuf[i, pl.ds(j * 16, 16)] = zvec
        return 0

    lax.fori_loop(0, _ZROWS * (D // 16), _zero_elem, 0)
    for k in range(_ROWS_TILE // _ZROWS):
        pltpu.sync_copy(zbuf, agg_sh.at[pl.ds(s * _ROWS_TILE + k * _ZROWS, _ZROWS)])
    plsc.subcore_barrier()

    pltpu.sync_copy(src_hbm.at[tile], src_v)
    pltpu.sync_copy(dst_hbm.at[tile], dst_v)

    def _edge_chunk(i, _):
        pltpu.async_copy(x_hbm.at[src_v.at[i]], rows_v, sem).wait()
        pltpu.sync_copy(rows_v, agg_sh.at[dst_v.at[i]], add=True)
        return 0

    lax.fori_loop(0, _NCHUNK, _edge_chunk, 0)
    plsc.subcore_barrier()

    r0 = s * _ROWS_TILE
    for k in range(_ROWS_TILE // _ZROWS):
        sl = pl.ds(r0 + k * _ZROWS, _ZROWS)
        pltpu.sync_copy(agg_sh.at[sl], out_hbm.at[c, sl])


import functools


@functools.cache
def _get_sc_agg():
    return pl.kernel(
        _sc_agg_body,
        out_type=jax.ShapeDtypeStruct((_NC, N_NODES, D), jnp.float32),
        mesh=plsc.VectorSubcoreMesh(core_axis_name="c", subcore_axis_name="s",
                                    num_cores=_NC, num_subcores=_NS),
        compiler_params=pltpu.CompilerParams(use_tc_tiling_on_sc=False),
        scratch_types=[
            pltpu.VMEM((_NCHUNK, _CHUNK), jnp.int32),
            pltpu.VMEM((_NCHUNK, _CHUNK), jnp.int32),
            pltpu.VMEM((_CHUNK, D), jnp.float32),
            pltpu.VMEM((_ZROWS, D), jnp.float32),
            pltpu.VMEM_SHARED((N_NODES, D), jnp.float32),
            pltpu.SemaphoreType.DMA,
        ],
    )


_BLK = 1000
_NBLK = N_NODES // _BLK


def _tc_body(x_ref, p_ref, w1_ref, w2_ref, b_ref, out_ref, loss_ref):
    a = jnp.clip(p_ref[0] + p_ref[1], 0.0, BOUND)
    x1 = (jnp.dot(x_ref[...], w1_ref[...], preferred_element_type=jnp.float32)
          + jnp.dot(a, w2_ref[...], preferred_element_type=jnp.float32)
          + b_ref[...])
    m = jnp.max(x1, axis=-1, keepdims=True)
    e = jnp.exp(x1 - m)
    y_soft = e / jnp.sum(e, axis=-1, keepdims=True)
    col = lax.broadcasted_iota(jnp.int32, x1.shape, 1)
    idx = jnp.min(jnp.where(x1 == m, col, OUT), axis=-1, keepdims=True)
    y_hard = (col == idx).astype(jnp.float32)
    x2 = y_soft + (y_hard - y_soft)
    out_ref[...] = x2

    @pl.when(pl.program_id(0) == 0)
    def _init():
        loss_ref[0, 0] = 0.0

    loss_ref[0, 0] += jnp.sum((x2 - x1) ** 2)


_tc = pl.pallas_call(
    _tc_body,
    grid=(_NBLK,),
    in_specs=[
        pl.BlockSpec((_BLK, D), lambda i: (i, 0)),
        pl.BlockSpec((_NC, _BLK, D), lambda i: (0, i, 0)),
        pl.BlockSpec((D, OUT), lambda i: (0, 0)),
        pl.BlockSpec((D, OUT), lambda i: (0, 0)),
        pl.BlockSpec((1, OUT), lambda i: (0, 0)),
    ],
    out_specs=[
        pl.BlockSpec((_BLK, OUT), lambda i: (i, 0)),
        pl.BlockSpec((1, 1), lambda i: (0, 0), memory_space=pltpu.SMEM),
    ],
    out_shape=[
        jax.ShapeDtypeStruct((N_NODES, OUT), jnp.float32),
        jax.ShapeDtypeStruct((1, 1), jnp.float32),
    ],
)


def kernel(x, edge_index, W, b):
    src = edge_index[0].reshape(_NW, _NCHUNK, _CHUNK)
    dst = edge_index[1].reshape(_NW, _NCHUNK, _CHUNK)
    partials = _get_sc_agg()(x, src, dst)
    x2, loss = _tc(x, partials, W[:D], W[D:], b.reshape(1, OUT))
    return x2, loss[0, 0] / (N_NODES * OUT)


# speedup vs baseline: 11.9214x; 1.5486x over previous
import jax
import jax.numpy as jnp
from jax import lax
from jax.experimental import pallas as pl
from jax.experimental.pallas import tpu as pltpu
from jax.experimental.pallas import tpu_sc as plsc

N_NODES = 10000
N_EDGES = 320000
D = 128
OUT = 128
BOUND = 10.0

_NC = 2
_NS = 16
_NW = _NC * _NS
_E_TILE = N_EDGES // _NW
_CHUNK = 80
_NCHUNK = _E_TILE // _CHUNK
_ZROWS = 125
_ROWS_TILE = N_NODES // _NS


def _sc_agg_body(x_hbm, src_hbm, dst_hbm, out_hbm,
                 src_v, dst_v, rows_a, rows_b, agg_sh, sem_a, sem_b):
    c = lax.axis_index("c")
    s = lax.axis_index("s")
    tile = c * _NS + s

    zvec = jnp.zeros((16,), jnp.float32)

    def _zero_elem(k, _):
        i = k // (D // 16)
        j = k % (D // 16)
        rows_a[i, pl.ds(j * 16, 16)] = zvec
        return 0

    lax.fori_loop(0, _CHUNK * (D // 16), _zero_elem, 0)
    r0 = s * _ROWS_TILE
    for k in range(_ROWS_TILE // _CHUNK):
        pltpu.sync_copy(rows_a, agg_sh.at[pl.ds(r0 + k * _CHUNK, _CHUNK)])
    _rem = _ROWS_TILE % _CHUNK
    pltpu.sync_copy(rows_a.at[pl.ds(0, _rem)],
                    agg_sh.at[pl.ds(r0 + _ROWS_TILE - _rem, _rem)])
    plsc.subcore_barrier()

    pltpu.sync_copy(src_hbm.at[tile], src_v)
    pltpu.sync_copy(dst_hbm.at[tile], dst_v)

    pltpu.async_copy(x_hbm.at[src_v.at[0]], rows_a, sem_a)

    def _edge_pair(k, _):
        i = 2 * k
        pltpu.async_copy(x_hbm.at[src_v.at[i + 1]], rows_b, sem_b)
        pltpu.make_async_copy(x_hbm.at[src_v.at[i]], rows_a, sem_a).wait()
        pltpu.sync_copy(rows_a, agg_sh.at[dst_v.at[i]], add=True)
        pltpu.async_copy(x_hbm.at[src_v.at[i + 2]], rows_a, sem_a)
        pltpu.make_async_copy(x_hbm.at[src_v.at[i + 1]], rows_b, sem_b).wait()
        pltpu.sync_copy(rows_b, agg_sh.at[dst_v.at[i + 1]], add=True)
        return 0

    lax.fori_loop(0, (_NCHUNK - 1) // 2, _edge_pair, 0)
    pltpu.make_async_copy(x_hbm.at[src_v.at[_NCHUNK - 1]], rows_a, sem_a).wait()
    pltpu.sync_copy(rows_a, agg_sh.at[dst_v.at[_NCHUNK - 1]], add=True)
    plsc.subcore_barrier()

    r0 = s * _ROWS_TILE
    for k in range(_ROWS_TILE // _ZROWS):
        sl = pl.ds(r0 + k * _ZROWS, _ZROWS)
        pltpu.sync_copy(agg_sh.at[sl], out_hbm.at[c, sl])


import functools


@functools.cache
def _get_sc_agg():
    return pl.kernel(
        _sc_agg_body,
        out_type=jax.ShapeDtypeStruct((_NC, N_NODES, D), jnp.float32),
        mesh=plsc.VectorSubcoreMesh(core_axis_name="c", subcore_axis_name="s",
                                    num_cores=_NC, num_subcores=_NS),
        compiler_params=pltpu.CompilerParams(use_tc_tiling_on_sc=False),
        scratch_types=[
            pltpu.VMEM((_NCHUNK, _CHUNK), jnp.int32),
            pltpu.VMEM((_NCHUNK, _CHUNK), jnp.int32),
            pltpu.VMEM((_CHUNK, D), jnp.float32),
            pltpu.VMEM((_CHUNK, D), jnp.float32),
            pltpu.VMEM_SHARED((N_NODES, D), jnp.float32),
            pltpu.SemaphoreType.DMA,
            pltpu.SemaphoreType.DMA,
        ],
    )


_BLK = 1000
_NBLK = N_NODES // _BLK


def _tc_body(x_ref, p_ref, w1_ref, w2_ref, b_ref, out_ref, loss_ref):
    a = jnp.clip(p_ref[0] + p_ref[1], 0.0, BOUND)
    x1 = (jnp.dot(x_ref[...], w1_ref[...], preferred_element_type=jnp.float32)
          + jnp.dot(a, w2_ref[...], preferred_element_type=jnp.float32)
          + b_ref[...])
    m = jnp.max(x1, axis=-1, keepdims=True)
    e = jnp.exp(x1 - m)
    y_soft = e / jnp.sum(e, axis=-1, keepdims=True)
    col = lax.broadcasted_iota(jnp.int32, x1.shape, 1)
    idx = jnp.min(jnp.where(x1 == m, col, OUT), axis=-1, keepdims=True)
    y_hard = (col == idx).astype(jnp.float32)
    x2 = y_soft + (y_hard - y_soft)
    out_ref[...] = x2

    @pl.when(pl.program_id(0) == 0)
    def _init():
        loss_ref[0, 0] = 0.0

    loss_ref[0, 0] += jnp.sum((x2 - x1) ** 2)


_tc = pl.pallas_call(
    _tc_body,
    grid=(_NBLK,),
    in_specs=[
        pl.BlockSpec((_BLK, D), lambda i: (i, 0)),
        pl.BlockSpec((_NC, _BLK, D), lambda i: (0, i, 0)),
        pl.BlockSpec((D, OUT), lambda i: (0, 0)),
        pl.BlockSpec((D, OUT), lambda i: (0, 0)),
        pl.BlockSpec((1, OUT), lambda i: (0, 0)),
    ],
    out_specs=[
        pl.BlockSpec((_BLK, OUT), lambda i: (i, 0)),
        pl.BlockSpec((1, 1), lambda i: (0, 0), memory_space=pltpu.SMEM),
    ],
    out_shape=[
        jax.ShapeDtypeStruct((N_NODES, OUT), jnp.float32),
        jax.ShapeDtypeStruct((1, 1), jnp.float32),
    ],
)


def kernel(x, edge_index, W, b):
    src = edge_index[0].reshape(_NW, _NCHUNK, _CHUNK)
    dst = edge_index[1].reshape(_NW, _NCHUNK, _CHUNK)
    partials = _get_sc_agg()(x, src, dst)
    x2, loss = _tc(x, partials, W[:D], W[D:], b.reshape(1, OUT))
    return x2, loss[0, 0] / (N_NODES * OUT)
